# speedup vs baseline: 7.4667x; 7.4667x over previous
import jax
import jax.numpy as jnp
from jax import lax
from jax.experimental import pallas as pl
from jax.experimental.pallas import tpu as pltpu
from jax.experimental.pallas import tpu_sc as plsc

N = 10000
E = 320000
D = 128

NC = 2
NS = 16
NW = NC * NS
EPW = E // NW
K = 80
C = EPW // K
N_PAD = 10240
RPW = N_PAD // NS


def _sc_aggregate_body(x_hbm, src_hbm, dst_hbm, agg_out, cnt_out,
                       src_v, dst_v, rows_v, ones_v, shared_agg, shared_cnt,
                       sem):
    cid = lax.axis_index("c")
    sid = lax.axis_index("s")
    wid = cid * NS + sid

    def _zrow(r, carry):
        for c8 in range(D // 16):
            rows_v[r, pl.ds(c8 * 16, 16)] = jnp.zeros((16,), jnp.float32)
        return carry
    lax.fori_loop(0, K, _zrow, 0)
    for i in range(K // 16):
        ones_v[pl.ds(i * 16, 16)] = jnp.ones((16,), jnp.float32)

    base = sid * RPW
    for m in range(RPW // K):
        pltpu.sync_copy(rows_v, shared_agg.at[pl.ds(base + m * K, K)])
    for m in range(RPW // D):
        pltpu.sync_copy(rows_v.at[0], shared_cnt.at[pl.ds(base + m * D, D)])
    plsc.subcore_barrier()

    pltpu.sync_copy(src_hbm.at[wid], src_v)
    pltpu.sync_copy(dst_hbm.at[wid], dst_v)

    def _chunk(j, carry):
        pltpu.async_copy(x_hbm.at[src_v.at[j]], rows_v, sem).wait()
        pltpu.sync_copy(rows_v, shared_agg.at[dst_v.at[j]], add=True)
        pltpu.sync_copy(ones_v, shared_cnt.at[dst_v.at[j]], add=True)
        return carry
    lax.fori_loop(0, C, _chunk, 0)
    plsc.subcore_barrier()

    pltpu.sync_copy(shared_agg.at[pl.ds(base, RPW)],
                    agg_out.at[cid, pl.ds(base, RPW)])
    pltpu.sync_copy(shared_cnt.at[pl.ds(base, RPW)],
                    cnt_out.at[cid, pl.ds(base, RPW)])


def _sc_aggregate(x, src, dst):
    mesh = plsc.VectorSubcoreMesh(core_axis_name="c", subcore_axis_name="s")
    kern = pl.kernel(
        _sc_aggregate_body,
        out_type=[
            jax.ShapeDtypeStruct((NC, N_PAD, D), jnp.float32),
            jax.ShapeDtypeStruct((NC, N_PAD), jnp.float32),
        ],
        mesh=mesh,
        scratch_types=[
            pltpu.VMEM((C, K), jnp.int32),
            pltpu.VMEM((C, K), jnp.int32),
            pltpu.VMEM((K, D), jnp.float32),
            pltpu.VMEM((K,), jnp.float32),
            pltpu.VMEM_SHARED((N_PAD, D), jnp.float32),
            pltpu.VMEM_SHARED((N_PAD,), jnp.float32),
            pltpu.SemaphoreType.DMA,
        ],
    )
    return kern(x, src, dst)


def _tc_layer1_body(p0, p1, c0, c1, x, wl, bl, wr, o_ref):
    cnt = jnp.maximum(c0[...] + c1[...], 1.0)
    mean = (p0[...] + p1[...]) / cnt
    h = (jnp.dot(mean, wl[...], preferred_element_type=jnp.float32)
         + bl[...]
         + jnp.dot(x[...], wr[...], preferred_element_type=jnp.float32))
    o_ref[...] = jnp.maximum(h, 0.0)


def _tc_layer2_body(p0, p1, c0, c1, x, wl, bl, wr, wlin, blin, o_ref):
    cnt = jnp.maximum(c0[...] + c1[...], 1.0)
    mean = (p0[...] + p1[...]) / cnt
    h = (jnp.dot(mean, wl[...], preferred_element_type=jnp.float32)
         + bl[...]
         + jnp.dot(x[...], wr[...], preferred_element_type=jnp.float32))
    h = jnp.maximum(h, 0.0)
    o_ref[...] = (jnp.dot(h, wlin[...], preferred_element_type=jnp.float32)
                  + blin[...])


_BN = 2048


def _row_spec():
    return pl.BlockSpec((_BN, D), lambda i: (i, 0))


def _cnt_spec():
    return pl.BlockSpec((_BN, 1), lambda i: (i, 0))


def _w_spec():
    return pl.BlockSpec((D, D), lambda i: (0, 0))


def _b_spec():
    return pl.BlockSpec((1, D), lambda i: (0, 0))


def _tc_layer1(p0, p1, c0, c1, x, wl, bl, wr):
    return pl.pallas_call(
        _tc_layer1_body,
        grid=(N_PAD // _BN,),
        in_specs=[_row_spec(), _row_spec(), _cnt_spec(), _cnt_spec(),
                  _row_spec(), _w_spec(), _b_spec(), _w_spec()],
        out_specs=_row_spec(),
        out_shape=jax.ShapeDtypeStruct((N_PAD, D), jnp.float32),
    )(p0, p1, c0, c1, x, wl, bl, wr)


def _tc_layer2(p0, p1, c0, c1, x, wl, bl, wr, wlin, blin):
    return pl.pallas_call(
        _tc_layer2_body,
        grid=(N_PAD // _BN,),
        in_specs=[_row_spec(), _row_spec(), _cnt_spec(), _cnt_spec(),
                  _row_spec(), _w_spec(), _b_spec(), _w_spec(),
                  _w_spec(), _b_spec()],
        out_specs=_row_spec(),
        out_shape=jax.ShapeDtypeStruct((N_PAD, D), jnp.float32),
    )(p0, p1, c0, c1, x, wl, bl, wr, wlin, blin)


def kernel(x, edge_index, W1l, b1l, W1r, W2l, b2l, W2r, Wlin, blin):
    x = x.astype(jnp.float32)
    src = edge_index[0].reshape(NW, C, K)
    dst = edge_index[1].reshape(NW, C, K)

    agg1, cnt1 = _sc_aggregate(x, src, dst)
    c0 = cnt1[0][:, None]
    c1 = cnt1[1][:, None]

    xp = jnp.pad(x, ((0, N_PAD - N), (0, 0)))
    h = _tc_layer1(agg1[0], agg1[1], c0, c1, xp,
                   W1l, b1l[None, :], W1r)

    agg2, _ = _sc_aggregate(h, src, dst)
    out = _tc_layer2(agg2[0], agg2[1], c0, c1, h,
                     W2l, b2l[None, :], W2r, Wlin, blin[None, :])
    return out[:N]


# speedup vs baseline: 8.3817x; 1.1225x over previous
import jax
import jax.numpy as jnp
from jax import lax
from jax.experimental import pallas as pl
from jax.experimental.pallas import tpu as pltpu
from jax.experimental.pallas import tpu_sc as plsc

N = 10000
E = 320000
D = 128
DH = D // 2

NC = 2
NS = 16
EPW = E // NS
K = 80
C = EPW // K
N_PAD = 10240
RPW = N_PAD // NS


def _sc_aggregate_body(x_hbm, src_hbm, dst_hbm, agg_out, cnt_out,
                       src_v, dst_v, rows0_v, rows1_v, ones_v,
                       shared_agg, shared_cnt, sem0, sem1):
    cid = lax.axis_index("c")
    sid = lax.axis_index("s")

    def _zrow(r, carry):
        for c16 in range(DH // 16):
            rows0_v[r, pl.ds(c16 * 16, 16)] = jnp.zeros((16,), jnp.float32)
        return carry
    lax.fori_loop(0, K, _zrow, 0)
    for i in range(K // 16):
        ones_v[pl.ds(i * 16, 16)] = jnp.ones((16,), jnp.float32)

    base = sid * RPW
    for m in range(RPW // K):
        pltpu.sync_copy(rows0_v, shared_agg.at[pl.ds(base + m * K, K)])

    @pl.when(cid == 0)
    def _():
        for m in range(RPW // DH):
            pltpu.sync_copy(rows0_v.at[0],
                            shared_cnt.at[pl.ds(base + m * DH, DH)])
    plsc.subcore_barrier()

    pltpu.sync_copy(src_hbm.at[sid], src_v)
    pltpu.sync_copy(dst_hbm.at[sid], dst_v)

    off_v = jnp.zeros((16,), jnp.int32) + cid * N_PAD

    def _off(r, carry):
        for c16 in range(K // 16):
            sl = src_v[r, pl.ds(c16 * 16, 16)]
            src_v[r, pl.ds(c16 * 16, 16)] = sl + off_v
        return carry
    lax.fori_loop(0, C, _off, 0)

    pltpu.async_copy(x_hbm.at[src_v.at[0]], rows0_v, sem0)

    def _chunk(j, carry):
        nxt = j + 1

        @pl.when(j % 2 == 0)
        def _():
            @pl.when(nxt < C)
            def _():
                pltpu.async_copy(x_hbm.at[src_v.at[nxt]], rows1_v, sem1)
            pltpu.make_async_copy(x_hbm.at[src_v.at[j]], rows0_v, sem0).wait()
            pltpu.sync_copy(rows0_v, shared_agg.at[dst_v.at[j]], add=True)

        @pl.when(j % 2 == 1)
        def _():
            @pl.when(nxt < C)
            def _():
                pltpu.async_copy(x_hbm.at[src_v.at[nxt]], rows0_v, sem0)
            pltpu.make_async_copy(x_hbm.at[src_v.at[j]], rows1_v, sem1).wait()
            pltpu.sync_copy(rows1_v, shared_agg.at[dst_v.at[j]], add=True)

        @pl.when(cid == 0)
        def _():
            pltpu.sync_copy(ones_v, shared_cnt.at[dst_v.at[j]], add=True)
        return carry
    lax.fori_loop(0, C, _chunk, 0)
    plsc.subcore_barrier()

    pltpu.sync_copy(shared_agg.at[pl.ds(base, RPW)],
                    agg_out.at[cid, pl.ds(base, RPW)])

    @pl.when(cid == 0)
    def _():
        pltpu.sync_copy(shared_cnt.at[pl.ds(base, RPW)],
                        cnt_out.at[pl.ds(base, RPW)])


def _sc_aggregate(xs, src, dst):
    mesh = plsc.VectorSubcoreMesh(core_axis_name="c", subcore_axis_name="s")
    kern = pl.kernel(
        _sc_aggregate_body,
        out_type=[
            jax.ShapeDtypeStruct((NC, N_PAD, DH), jnp.float32),
            jax.ShapeDtypeStruct((N_PAD,), jnp.float32),
        ],
        mesh=mesh,
        scratch_types=[
            pltpu.VMEM((C, K), jnp.int32),
            pltpu.VMEM((C, K), jnp.int32),
            pltpu.VMEM((K, DH), jnp.float32),
            pltpu.VMEM((K, DH), jnp.float32),
            pltpu.VMEM((K,), jnp.float32),
            pltpu.VMEM_SHARED((N_PAD, DH), jnp.float32),
            pltpu.VMEM_SHARED((N_PAD,), jnp.float32),
            pltpu.SemaphoreType.DMA,
            pltpu.SemaphoreType.DMA,
        ],
        compiler_params=pltpu.CompilerParams(use_tc_tiling_on_sc=False),
    )
    return kern(xs, src, dst)


def _tc_layer1_body(p0, p1, c0, x, wl, bl, wr, o_ref):
    cnt = jnp.maximum(c0[...], 1.0)
    mean = jnp.concatenate([p0[...], p1[...]], axis=1) / cnt
    h = (jnp.dot(mean, wl[...], preferred_element_type=jnp.float32)
         + bl[...]
         + jnp.dot(x[...], wr[...], preferred_element_type=jnp.float32))
    o_ref[...] = jnp.maximum(h, 0.0)


def _tc_layer2_body(p0, p1, c0, x, wl, bl, wr, wlin, blin, o_ref):
    cnt = jnp.maximum(c0[...], 1.0)
    mean = jnp.concatenate([p0[...], p1[...]], axis=1) / cnt
    h = (jnp.dot(mean, wl[...], preferred_element_type=jnp.float32)
         + bl[...]
         + jnp.dot(x[...], wr[...], preferred_element_type=jnp.float32))
    h = jnp.maximum(h, 0.0)
    o_ref[...] = (jnp.dot(h, wlin[...], preferred_element_type=jnp.float32)
                  + blin[...])


_BN = 2048


def _row_spec():
    return pl.BlockSpec((_BN, D), lambda i: (i, 0))


def _half_spec():
    return pl.BlockSpec((_BN, DH), lambda i: (i, 0))


def _cnt_spec():
    return pl.BlockSpec((_BN, 1), lambda i: (i, 0))


def _w_spec():
    return pl.BlockSpec((D, D), lambda i: (0, 0))


def _b_spec():
    return pl.BlockSpec((1, D), lambda i: (0, 0))


def _tc_layer1(p0, p1, c0, x, wl, bl, wr):
    return pl.pallas_call(
        _tc_layer1_body,
        grid=(N_PAD // _BN,),
        in_specs=[_half_spec(), _half_spec(), _cnt_spec(),
                  _row_spec(), _w_spec(), _b_spec(), _w_spec()],
        out_specs=_row_spec(),
        out_shape=jax.ShapeDtypeStruct((N_PAD, D), jnp.float32),
    )(p0, p1, c0, x, wl, bl, wr)


def _tc_layer2(p0, p1, c0, x, wl, bl, wr, wlin, blin):
    return pl.pallas_call(
        _tc_layer2_body,
        grid=(N_PAD // _BN,),
        in_specs=[_half_spec(), _half_spec(), _cnt_spec(),
                  _row_spec(), _w_spec(), _b_spec(), _w_spec(),
                  _w_spec(), _b_spec()],
        out_specs=_row_spec(),
        out_shape=jax.ShapeDtypeStruct((N_PAD, D), jnp.float32),
    )(p0, p1, c0, x, wl, bl, wr, wlin, blin)


def _split_stack(a):
    return jnp.concatenate([a[:, :DH], a[:, DH:]], axis=0)


def kernel(x, edge_index, W1l, b1l, W1r, W2l, b2l, W2r, Wlin, blin):
    x = x.astype(jnp.float32)
    src = edge_index[0].reshape(NS, C, K)
    dst = edge_index[1].reshape(NS, C, K)

    xp = jnp.pad(x, ((0, N_PAD - N), (0, 0)))
    agg1, cnt1 = _sc_aggregate(_split_stack(xp), src, dst)
    c0 = cnt1[:, None]

    h = _tc_layer1(agg1[0], agg1[1], c0, xp, W1l, b1l[None, :], W1r)

    agg2, _ = _sc_aggregate(_split_stack(h), src, dst)
    out = _tc_layer2(agg2[0], agg2[1], c0, h,
                     W2l, b2l[None, :], W2r, Wlin, blin[None, :])
    return out[:N]


# speedup vs baseline: 10.5486x; 1.2585x over previous
import jax
import jax.numpy as jnp
from jax import lax
from jax.experimental import pallas as pl
from jax.experimental.pallas import tpu as pltpu
from jax.experimental.pallas import tpu_sc as plsc

N = 10000
E = 320000
D = 128
DH = D // 2

NC = 2
NS = 16
EPW = E // NS
K = 128
EPW_PAD = 20096
C = EPW_PAD // K
PADE = EPW_PAD - EPW
N_PAD = 10240
RPW = N_PAD // NS


def _sc_aggregate_body(x_hbm, src_hbm, dst_hbm, agg_out, cnt_out,
                       src_v, dst_v, rows0_v, rows1_v, ones_v,
                       shared_agg, shared_cnt, sem0, sem1):
    cid = lax.axis_index("c")
    sid = lax.axis_index("s")

    def _zrow(r, carry):
        for c16 in range(DH // 16):
            rows0_v[r, pl.ds(c16 * 16, 16)] = jnp.zeros((16,), jnp.float32)
        return carry
    lax.fori_loop(0, K, _zrow, 0)
    for i in range(K // 16):
        ones_v[pl.ds(i * 16, 16)] = jnp.ones((16,), jnp.float32)

    base = sid * RPW
    for m in range(RPW // K):
        pltpu.sync_copy(rows0_v, shared_agg.at[pl.ds(base + m * K, K)])
    for m in range(RPW // DH):
        pltpu.sync_copy(rows0_v.at[0],
                        shared_cnt.at[pl.ds(base + m * DH, DH)])
    plsc.subcore_barrier()

    pltpu.sync_copy(src_hbm.at[sid], src_v)
    pltpu.sync_copy(dst_hbm.at[sid], dst_v)

    cid_v = jnp.zeros((16,), jnp.int32) + cid

    def _off(r, carry):
        for c16 in range(K // 16):
            sl = src_v[r, pl.ds(c16 * 16, 16)]
            src_v[r, pl.ds(c16 * 16, 16)] = sl + sl + cid_v
        return carry
    lax.fori_loop(0, C, _off, 0)

    pltpu.async_copy(x_hbm.at[src_v.at[0]], rows0_v, sem0)

    def _chunk(j, carry):
        nxt = j + 1

        @pl.when(j % 2 == 0)
        def _():
            @pl.when(nxt < C)
            def _():
                pltpu.async_copy(x_hbm.at[src_v.at[nxt]], rows1_v, sem1)
            pltpu.make_async_copy(x_hbm.at[src_v.at[j]], rows0_v, sem0).wait()
            pltpu.sync_copy(rows0_v, shared_agg.at[dst_v.at[j]], add=True)

            @pl.when(cid == 0)
            def _():
                pltpu.sync_copy(ones_v, shared_cnt.at[dst_v.at[j]], add=True)

        @pl.when(j % 2 == 1)
        def _():
            @pl.when(nxt < C)
            def _():
                pltpu.async_copy(x_hbm.at[src_v.at[nxt]], rows0_v, sem0)
            pltpu.make_async_copy(x_hbm.at[src_v.at[j]], rows1_v, sem1).wait()
            pltpu.sync_copy(rows1_v, shared_agg.at[dst_v.at[j]], add=True)

            @pl.when(cid == 1)
            def _():
                pltpu.sync_copy(ones_v, shared_cnt.at[dst_v.at[j]], add=True)
        return carry
    lax.fori_loop(0, C, _chunk, 0)
    plsc.subcore_barrier()

    pltpu.sync_copy(shared_agg.at[pl.ds(base, RPW)],
                    agg_out.at[cid, pl.ds(base, RPW)])
    pltpu.sync_copy(shared_cnt.at[pl.ds(base, RPW)],
                    cnt_out.at[cid, pl.ds(base, RPW)])


def _sc_aggregate(xs, src, dst):
    mesh = plsc.VectorSubcoreMesh(core_axis_name="c", subcore_axis_name="s")
    kern = pl.kernel(
        _sc_aggregate_body,
        out_type=[
            jax.ShapeDtypeStruct((NC, N_PAD, DH), jnp.float32),
            jax.ShapeDtypeStruct((NC, N_PAD), jnp.float32),
        ],
        mesh=mesh,
        scratch_types=[
            pltpu.VMEM((C, K), jnp.int32),
            pltpu.VMEM((C, K), jnp.int32),
            pltpu.VMEM((K, DH), jnp.float32),
            pltpu.VMEM((K, DH), jnp.float32),
            pltpu.VMEM((K,), jnp.float32),
            pltpu.VMEM_SHARED((N_PAD, DH), jnp.float32),
            pltpu.VMEM_SHARED((N_PAD,), jnp.float32),
            pltpu.SemaphoreType.DMA,
            pltpu.SemaphoreType.DMA,
        ],
        compiler_params=pltpu.CompilerParams(use_tc_tiling_on_sc=False),
    )
    return kern(xs, src, dst)


def _tc_layer1_body(p0, p1, c0, c1, x, wl, bl, wr, o_ref):
    cnt = jnp.maximum(c0[...] + c1[...], 1.0)
    mean = jnp.concatenate([p0[...], p1[...]], axis=1) / cnt
    h = (jnp.dot(mean, wl[...], preferred_element_type=jnp.float32)
         + bl[...]
         + jnp.dot(x[...], wr[...], preferred_element_type=jnp.float32))
    o_ref[...] = jnp.maximum(h, 0.0)


def _tc_layer2_body(p0, p1, c0, c1, x, wl, bl, wr, wlin, blin, o_ref):
    cnt = jnp.maximum(c0[...] + c1[...], 1.0)
    mean = jnp.concatenate([p0[...], p1[...]], axis=1) / cnt
    h = (jnp.dot(mean, wl[...], preferred_element_type=jnp.float32)
         + bl[...]
         + jnp.dot(x[...], wr[...], preferred_element_type=jnp.float32))
    h = jnp.maximum(h, 0.0)
    o_ref[...] = (jnp.dot(h, wlin[...], preferred_element_type=jnp.float32)
                  + blin[...])


_BN = 2048


def _row_spec():
    return pl.BlockSpec((_BN, D), lambda i: (i, 0))


def _half_spec():
    return pl.BlockSpec((_BN, DH), lambda i: (i, 0))


def _cnt_spec():
    return pl.BlockSpec((_BN, 1), lambda i: (i, 0))


def _w_spec():
    return pl.BlockSpec((D, D), lambda i: (0, 0))


def _b_spec():
    return pl.BlockSpec((1, D), lambda i: (0, 0))


def _tc_layer1(p0, p1, c0, c1, x, wl, bl, wr):
    return pl.pallas_call(
        _tc_layer1_body,
        grid=(N_PAD // _BN,),
        in_specs=[_half_spec(), _half_spec(), _cnt_spec(), _cnt_spec(),
                  _row_spec(), _w_spec(), _b_spec(), _w_spec()],
        out_specs=_row_spec(),
        out_shape=jax.ShapeDtypeStruct((N_PAD, D), jnp.float32),
    )(p0, p1, c0, c1, x, wl, bl, wr)


def _tc_layer2(p0, p1, c0, c1, x, wl, bl, wr, wlin, blin):
    return pl.pallas_call(
        _tc_layer2_body,
        grid=(N_PAD // _BN,),
        in_specs=[_half_spec(), _half_spec(), _cnt_spec(), _cnt_spec(),
                  _row_spec(), _w_spec(), _b_spec(), _w_spec(),
                  _w_spec(), _b_spec()],
        out_specs=_row_spec(),
        out_shape=jax.ShapeDtypeStruct((N_PAD, D), jnp.float32),
    )(p0, p1, c0, c1, x, wl, bl, wr, wlin, blin)


def kernel(x, edge_index, W1l, b1l, W1r, W2l, b2l, W2r, Wlin, blin):
    x = x.astype(jnp.float32)

    srcf = edge_index[0].reshape(NS, EPW)
    dstf = edge_index[1].reshape(NS, EPW)
    pad_pos = jnp.arange(NS * PADE, dtype=jnp.int32).reshape(NS, PADE)
    pad_src = (pad_pos * 613) % N
    pad_dst = N + (pad_pos % (N_PAD - N))
    src = jnp.concatenate([srcf, pad_src], axis=1).reshape(NS, C, K)
    dst = jnp.concatenate([dstf, pad_dst], axis=1).reshape(NS, C, K)

    xp = jnp.pad(x, ((0, N_PAD - N), (0, 0)))
    agg1, cnt1 = _sc_aggregate(xp.reshape(2 * N_PAD, DH), src, dst)
    c0 = cnt1[0][:, None]
    c1 = cnt1[1][:, None]

    h = _tc_layer1(agg1[0], agg1[1], c0, c1, xp, W1l, b1l[None, :], W1r)

    agg2, _ = _sc_aggregate(h.reshape(2 * N_PAD, DH), src, dst)
    out = _tc_layer2(agg2[0], agg2[1], c0, c1, h,
                     W2l, b2l[None, :], W2r, Wlin, blin[None, :])
    return out[:N]


# speedup vs baseline: 10.9816x; 1.0410x over previous
import jax
import jax.numpy as jnp
from jax import lax
from jax.experimental import pallas as pl
from jax.experimental.pallas import tpu as pltpu
from jax.experimental.pallas import tpu_sc as plsc

N = 10000
E = 320000
D = 128
DH = D // 2

NC = 2
NS = 16
EPW = E // NS
K = 128
EPW_PAD = 20096
C = EPW_PAD // K
PADE = EPW_PAD - EPW
N_PAD = 10240
RPW = N_PAD // NS


def _sc_aggregate_body(x_hbm, src0_hbm, src1_hbm, dst_hbm, agg_out, cnt_out,
                       src_v, dst_v, rows0_v, rows1_v, ones_v,
                       shared_agg, shared_cnt, sem0, sem1, csem):
    cid = lax.axis_index("c")
    sid = lax.axis_index("s")

    def _zrow(r, carry):
        for c16 in range(DH // 16):
            rows0_v[r, pl.ds(c16 * 16, 16)] = jnp.zeros((16,), jnp.float32)
        return carry
    lax.fori_loop(0, K, _zrow, 0)
    for i in range(K // 16):
        ones_v[pl.ds(i * 16, 16)] = jnp.ones((16,), jnp.float32)

    base = sid * RPW
    for m in range(RPW // K):
        pltpu.sync_copy(rows0_v, shared_agg.at[pl.ds(base + m * K, K)])
    for m in range(RPW // DH):
        pltpu.sync_copy(rows0_v.at[0],
                        shared_cnt.at[pl.ds(base + m * DH, DH)])
    plsc.subcore_barrier()

    @pl.when(cid == 0)
    def _():
        pltpu.sync_copy(src0_hbm.at[sid], src_v)

    @pl.when(cid == 1)
    def _():
        pltpu.sync_copy(src1_hbm.at[sid], src_v)
    pltpu.sync_copy(dst_hbm.at[sid], dst_v)

    pltpu.async_copy(x_hbm.at[src_v.at[0]], rows0_v, sem0)

    def _chunk(j, carry):
        nxt = j + 1

        @pl.when(j % 2 == 0)
        def _():
            @pl.when(nxt < C)
            def _():
                pltpu.async_copy(x_hbm.at[src_v.at[nxt]], rows1_v, sem1)
            pltpu.make_async_copy(x_hbm.at[src_v.at[j]], rows0_v, sem0).wait()

            @pl.when(cid == 0)
            def _():
                pltpu.async_copy(ones_v, shared_cnt.at[dst_v.at[j]], csem,
                                 add=True)
            pltpu.sync_copy(rows0_v, shared_agg.at[dst_v.at[j]], add=True)

        @pl.when(j % 2 == 1)
        def _():
            @pl.when(nxt < C)
            def _():
                pltpu.async_copy(x_hbm.at[src_v.at[nxt]], rows0_v, sem0)
            pltpu.make_async_copy(x_hbm.at[src_v.at[j]], rows1_v, sem1).wait()

            @pl.when(cid == 1)
            def _():
                pltpu.async_copy(ones_v, shared_cnt.at[dst_v.at[j]], csem,
                                 add=True)
            pltpu.sync_copy(rows1_v, shared_agg.at[dst_v.at[j]], add=True)
        return carry
    lax.fori_loop(0, C, _chunk, 0)

    n_counts = jnp.where(cid == 0, (C + 1) // 2, C // 2)

    def _drain(i, carry):
        pltpu.make_async_copy(ones_v, shared_cnt.at[dst_v.at[0]],
                              csem).wait()
        return carry
    lax.fori_loop(0, n_counts, _drain, 0)
    plsc.subcore_barrier()

    pltpu.sync_copy(shared_agg.at[pl.ds(base, RPW)],
                    agg_out.at[cid, pl.ds(base, RPW)])
    pltpu.sync_copy(shared_cnt.at[pl.ds(base, RPW)],
                    cnt_out.at[cid, pl.ds(base, RPW)])


def _sc_aggregate(xs, src0, src1, dst):
    mesh = plsc.VectorSubcoreMesh(core_axis_name="c", subcore_axis_name="s")
    kern = pl.kernel(
        _sc_aggregate_body,
        out_type=[
            jax.ShapeDtypeStruct((NC, N_PAD, DH), jnp.float32),
            jax.ShapeDtypeStruct((NC, N_PAD), jnp.float32),
        ],
        mesh=mesh,
        scratch_types=[
            pltpu.VMEM((C, K), jnp.int32),
            pltpu.VMEM((C, K), jnp.int32),
            pltpu.VMEM((K, DH), jnp.float32),
            pltpu.VMEM((K, DH), jnp.float32),
            pltpu.VMEM((K,), jnp.float32),
            pltpu.VMEM_SHARED((N_PAD, DH), jnp.float32),
            pltpu.VMEM_SHARED((N_PAD,), jnp.float32),
            pltpu.SemaphoreType.DMA,
            pltpu.SemaphoreType.DMA,
            pltpu.SemaphoreType.DMA,
        ],
        compiler_params=pltpu.CompilerParams(use_tc_tiling_on_sc=False),
    )
    return kern(xs, src0, src1, dst)


def _tc_layer1_body(p0, p1, c0, c1, x, wl, bl, wr, o_ref):
    cnt = jnp.maximum(c0[...] + c1[...], 1.0)
    mean = jnp.concatenate([p0[...], p1[...]], axis=1) / cnt
    h = (jnp.dot(mean, wl[...], preferred_element_type=jnp.float32)
         + bl[...]
         + jnp.dot(x[...], wr[...], preferred_element_type=jnp.float32))
    o_ref[...] = jnp.maximum(h, 0.0)


def _tc_layer2_body(p0, p1, c0, c1, x, wl, bl, wr, wlin, blin, o_ref):
    cnt = jnp.maximum(c0[...] + c1[...], 1.0)
    mean = jnp.concatenate([p0[...], p1[...]], axis=1) / cnt
    h = (jnp.dot(mean, wl[...], preferred_element_type=jnp.float32)
         + bl[...]
         + jnp.dot(x[...], wr[...], preferred_element_type=jnp.float32))
    h = jnp.maximum(h, 0.0)
    o_ref[...] = (jnp.dot(h, wlin[...], preferred_element_type=jnp.float32)
                  + blin[...])


_BN = 2048


def _row_spec():
    return pl.BlockSpec((_BN, D), lambda i: (i, 0))


def _half_spec():
    return pl.BlockSpec((_BN, DH), lambda i: (i, 0))


def _cnt_spec():
    return pl.BlockSpec((_BN, 1), lambda i: (i, 0))


def _w_spec():
    return pl.BlockSpec((D, D), lambda i: (0, 0))


def _b_spec():
    return pl.BlockSpec((1, D), lambda i: (0, 0))


def _tc_layer1(p0, p1, c0, c1, x, wl, bl, wr):
    return pl.pallas_call(
        _tc_layer1_body,
        grid=(N_PAD // _BN,),
        in_specs=[_half_spec(), _half_spec(), _cnt_spec(), _cnt_spec(),
                  _row_spec(), _w_spec(), _b_spec(), _w_spec()],
        out_specs=_row_spec(),
        out_shape=jax.ShapeDtypeStruct((N_PAD, D), jnp.float32),
    )(p0, p1, c0, c1, x, wl, bl, wr)


_BN2 = 2000


def _tc_layer2(p0, p1, c0, c1, x, wl, bl, wr, wlin, blin):
    row2 = pl.BlockSpec((_BN2, D), lambda i: (i, 0))
    half2 = pl.BlockSpec((_BN2, DH), lambda i: (i, 0))
    cnt2 = pl.BlockSpec((_BN2, 1), lambda i: (i, 0))
    return pl.pallas_call(
        _tc_layer2_body,
        grid=(N // _BN2,),
        in_specs=[half2, half2, cnt2, cnt2,
                  row2, _w_spec(), _b_spec(), _w_spec(),
                  _w_spec(), _b_spec()],
        out_specs=row2,
        out_shape=jax.ShapeDtypeStruct((N, D), jnp.float32),
    )(p0, p1, c0, c1, x, wl, bl, wr, wlin, blin)


def kernel(x, edge_index, W1l, b1l, W1r, W2l, b2l, W2r, Wlin, blin):
    x = x.astype(jnp.float32)

    srcf = edge_index[0].reshape(NS, EPW)
    dstf = edge_index[1].reshape(NS, EPW)
    pad_pos = jnp.arange(NS * PADE, dtype=jnp.int32).reshape(NS, PADE)
    pad_src = (pad_pos * 613) % N
    pad_dst = N + (pad_pos % (N_PAD - N))
    srcp = jnp.concatenate([srcf, pad_src], axis=1)
    src0 = (2 * srcp).reshape(NS, C, K)
    src1 = (2 * srcp + 1).reshape(NS, C, K)
    dst = jnp.concatenate([dstf, pad_dst], axis=1).reshape(NS, C, K)

    xp = jnp.pad(x, ((0, N_PAD - N), (0, 0)))
    agg1, cnt1 = _sc_aggregate(xp.reshape(2 * N_PAD, DH), src0, src1, dst)
    c0 = cnt1[0][:, None]
    c1 = cnt1[1][:, None]

    h = _tc_layer1(agg1[0], agg1[1], c0, c1, xp, W1l, b1l[None, :], W1r)

    agg2, _ = _sc_aggregate(h.reshape(2 * N_PAD, DH), src0, src1, dst)
    return _tc_layer2(agg2[0], agg2[1], c0, c1, h,
                      W2l, b2l[None, :], W2r, Wlin, blin[None, :])


# speedup vs baseline: 11.4087x; 1.0389x over previous
import jax
import jax.numpy as jnp
from jax import lax
from jax.experimental import pallas as pl
from jax.experimental.pallas import tpu as pltpu
from jax.experimental.pallas import tpu_sc as plsc

N = 10000
E = 320000
D = 128
DH = D // 2

NC = 2
NS = 16
EPW = E // NS
K = 128
EPW_PAD = 20096
C = EPW_PAD // K
PADE = EPW_PAD - EPW
N_PAD = 10240
RPW = N_PAD // NS


def _sc_aggregate_body(x_hbm, src0_hbm, src1_hbm, dst_hbm, agg_out, cnt_out,
                       src_v, dst_v, rows0_v, rows1_v, ones_v,
                       shared_agg, shared_cnt, sem0, sem1, csem):
    cid = lax.axis_index("c")
    sid = lax.axis_index("s")

    def _zrow(r, carry):
        for c16 in range(DH // 16):
            rows0_v[r, pl.ds(c16 * 16, 16)] = jnp.zeros((16,), jnp.float32)
        return carry
    lax.fori_loop(0, K, _zrow, 0)
    for i in range(K // 16):
        ones_v[pl.ds(i * 16, 16)] = jnp.ones((16,), jnp.float32)

    base = sid * RPW
    for m in range(RPW // K):
        pltpu.sync_copy(rows0_v, shared_agg.at[pl.ds(base + m * K, K)])
    for m in range(RPW // DH):
        pltpu.sync_copy(rows0_v.at[0],
                        shared_cnt.at[pl.ds(base + m * DH, DH)])
    plsc.subcore_barrier()

    @pl.when(cid == 0)
    def _():
        pltpu.sync_copy(src0_hbm.at[sid], src_v)

    @pl.when(cid == 1)
    def _():
        pltpu.sync_copy(src1_hbm.at[sid], src_v)
    pltpu.sync_copy(dst_hbm.at[sid], dst_v)

    pltpu.async_copy(x_hbm.at[src_v.at[0]], rows0_v, sem0)

    def _chunk(j, carry):
        nxt = j + 1

        @pl.when(j % 2 == 0)
        def _():
            @pl.when(nxt < C)
            def _():
                pltpu.async_copy(x_hbm.at[src_v.at[nxt]], rows1_v, sem1)
            pltpu.make_async_copy(x_hbm.at[src_v.at[j]], rows0_v, sem0).wait()

            @pl.when(cid == 0)
            def _():
                pltpu.async_copy(ones_v, shared_cnt.at[dst_v.at[j]], csem,
                                 add=True)
            pltpu.sync_copy(rows0_v, shared_agg.at[dst_v.at[j]], add=True)

        @pl.when(j % 2 == 1)
        def _():
            @pl.when(nxt < C)
            def _():
                pltpu.async_copy(x_hbm.at[src_v.at[nxt]], rows0_v, sem0)
            pltpu.make_async_copy(x_hbm.at[src_v.at[j]], rows1_v, sem1).wait()

            @pl.when(cid == 1)
            def _():
                pltpu.async_copy(ones_v, shared_cnt.at[dst_v.at[j]], csem,
                                 add=True)
            pltpu.sync_copy(rows1_v, shared_agg.at[dst_v.at[j]], add=True)
        return carry
    lax.fori_loop(0, C, _chunk, 0)

    n_counts = jnp.where(cid == 0, (C + 1) // 2, C // 2)

    def _drain(i, carry):
        pltpu.make_async_copy(ones_v, shared_cnt.at[dst_v.at[0]],
                              csem).wait()
        return carry
    lax.fori_loop(0, n_counts, _drain, 0)
    plsc.subcore_barrier()

    pltpu.sync_copy(shared_agg.at[pl.ds(base, RPW)],
                    agg_out.at[cid, pl.ds(base, RPW)])
    pltpu.sync_copy(shared_cnt.at[pl.ds(base, RPW)],
                    cnt_out.at[cid, pl.ds(base, RPW)])


def _sc_aggregate(xs, src0, src1, dst):
    mesh = plsc.VectorSubcoreMesh(core_axis_name="c", subcore_axis_name="s")
    kern = pl.kernel(
        _sc_aggregate_body,
        out_type=[
            jax.ShapeDtypeStruct((NC, N_PAD, DH), jnp.float32),
            jax.ShapeDtypeStruct((NC, N_PAD), jnp.float32),
        ],
        mesh=mesh,
        scratch_types=[
            pltpu.VMEM((C, K), jnp.int32),
            pltpu.VMEM((C, K), jnp.int32),
            pltpu.VMEM((K, DH), jnp.float32),
            pltpu.VMEM((K, DH), jnp.float32),
            pltpu.VMEM((K,), jnp.float32),
            pltpu.VMEM_SHARED((N_PAD, DH), jnp.float32),
            pltpu.VMEM_SHARED((N_PAD,), jnp.float32),
            pltpu.SemaphoreType.DMA,
            pltpu.SemaphoreType.DMA,
            pltpu.SemaphoreType.DMA,
        ],
        compiler_params=pltpu.CompilerParams(use_tc_tiling_on_sc=False),
    )
    return kern(xs, src0, src1, dst)


def _tc_layer1_body(p, c, x, wl, bl, wr, o_ref):
    cnt = jnp.maximum(c[0] + c[1], 1.0)
    mean = jnp.concatenate([p[0], p[1]], axis=1) / cnt
    h = (jnp.dot(mean, wl[...], preferred_element_type=jnp.float32)
         + bl[...]
         + jnp.dot(x[...], wr[...], preferred_element_type=jnp.float32))
    o_ref[...] = jnp.maximum(h, 0.0)


def _tc_layer2_body(p, c, x, wl, bl, wr, wlin, blin, o_ref):
    cnt = jnp.maximum(c[0] + c[1], 1.0)
    mean = jnp.concatenate([p[0], p[1]], axis=1) / cnt
    h = (jnp.dot(mean, wl[...], preferred_element_type=jnp.float32)
         + bl[...]
         + jnp.dot(x[...], wr[...], preferred_element_type=jnp.float32))
    h = jnp.maximum(h, 0.0)
    o_ref[...] = (jnp.dot(h, wlin[...], preferred_element_type=jnp.float32)
                  + blin[...])


_BN = 2000


def _row_spec():
    return pl.BlockSpec((_BN, D), lambda i: (i, 0))


def _agg_spec():
    return pl.BlockSpec((NC, _BN, DH), lambda i: (0, i, 0))


def _cnt_spec():
    return pl.BlockSpec((NC, _BN, 1), lambda i: (0, i, 0))


def _w_spec():
    return pl.BlockSpec((D, D), lambda i: (0, 0))


def _b_spec():
    return pl.BlockSpec((1, D), lambda i: (0, 0))


def _tc_layer1(p, c, x, wl, bl, wr):
    return pl.pallas_call(
        _tc_layer1_body,
        grid=(N // _BN,),
        in_specs=[_agg_spec(), _cnt_spec(),
                  _row_spec(), _w_spec(), _b_spec(), _w_spec()],
        out_specs=_row_spec(),
        out_shape=jax.ShapeDtypeStruct((N, D), jnp.float32),
    )(p, c, x, wl, bl, wr)


def _tc_layer2(p, c, x, wl, bl, wr, wlin, blin):
    return pl.pallas_call(
        _tc_layer2_body,
        grid=(N // _BN,),
        in_specs=[_agg_spec(), _cnt_spec(),
                  _row_spec(), _w_spec(), _b_spec(), _w_spec(),
                  _w_spec(), _b_spec()],
        out_specs=_row_spec(),
        out_shape=jax.ShapeDtypeStruct((N, D), jnp.float32),
    )(p, c, x, wl, bl, wr, wlin, blin)


def kernel(x, edge_index, W1l, b1l, W1r, W2l, b2l, W2r, Wlin, blin):
    x = x.astype(jnp.float32)

    srcf = edge_index[0].reshape(NS, EPW)
    dstf = edge_index[1].reshape(NS, EPW)
    pad_pos = jnp.arange(NS * PADE, dtype=jnp.int32).reshape(NS, PADE)
    pad_src = (pad_pos * 613) % N
    pad_dst = N + (pad_pos % (N_PAD - N))
    srcp = jnp.concatenate([srcf, pad_src], axis=1)
    src0 = (2 * srcp).reshape(NS, C, K)
    src1 = (2 * srcp + 1).reshape(NS, C, K)
    dst = jnp.concatenate([dstf, pad_dst], axis=1).reshape(NS, C, K)

    agg1, cnt1 = _sc_aggregate(x.reshape(2 * N, DH), src0, src1, dst)
    cnt1 = cnt1[..., None]

    h = _tc_layer1(agg1, cnt1, x, W1l, b1l[None, :], W1r)

    agg2, _ = _sc_aggregate(h.reshape(2 * N, DH), src0, src1, dst)
    return _tc_layer2(agg2, cnt1, h,
                      W2l, b2l[None, :], W2r, Wlin, blin[None, :])


# speedup vs baseline: 11.9933x; 1.0512x over previous
import jax
import jax.numpy as jnp
from jax import lax
from jax.experimental import pallas as pl
from jax.experimental.pallas import tpu as pltpu
from jax.experimental.pallas import tpu_sc as plsc

N = 10000
E = 320000
D = 128
DH = D // 2

NC = 2
NS = 16
EPW = E // NS
K = 128
EPW_PAD = 20096
C = EPW_PAD // K
PADE = EPW_PAD - EPW
N_PAD = 10240
RPW = N_PAD // NS


def _sc_aggregate_body(x_hbm, src0_hbm, src1_hbm, dst_hbm, agg_out, cnt_out,
                       src_v, dst_v, rows0_v, rows1_v, rows2_v, rows3_v,
                       ones_v, shared_agg, shared_cnt,
                       gsem0, gsem1, gsem2, gsem3,
                       ssem0, ssem1, ssem2, ssem3, csem):
    cid = lax.axis_index("c")
    sid = lax.axis_index("s")

    def _zrow(r, carry):
        for c16 in range(DH // 16):
            rows0_v[r, pl.ds(c16 * 16, 16)] = jnp.zeros((16,), jnp.float32)
        return carry
    lax.fori_loop(0, K, _zrow, 0)
    for i in range(K // 16):
        ones_v[pl.ds(i * 16, 16)] = jnp.ones((16,), jnp.float32)

    base = sid * RPW
    for m in range(RPW // K):
        pltpu.sync_copy(rows0_v, shared_agg.at[pl.ds(base + m * K, K)])
    for m in range(RPW // DH):
        pltpu.sync_copy(rows0_v.at[0],
                        shared_cnt.at[pl.ds(base + m * DH, DH)])
    plsc.subcore_barrier()

    @pl.when(cid == 0)
    def _():
        pltpu.sync_copy(src0_hbm.at[sid], src_v)

    @pl.when(cid == 1)
    def _():
        pltpu.sync_copy(src1_hbm.at[sid], src_v)
    pltpu.sync_copy(dst_hbm.at[sid], dst_v)

    rows = (rows0_v, rows1_v, rows2_v, rows3_v)
    gsem = (gsem0, gsem1, gsem2, gsem3)
    ssem = (ssem0, ssem1, ssem2, ssem3)
    pltpu.async_copy(x_hbm.at[src_v.at[0]], rows[0], gsem[0])
    pltpu.async_copy(x_hbm.at[src_v.at[1]], rows[1], gsem[1])

    def _chunk(j, carry):
        for b in range(4):
            @pl.when(j % 4 == b)
            def _(b=b):
                b2 = (b + 2) % 4
                pltpu.make_async_copy(x_hbm.at[src_v.at[j]], rows[b],
                                      gsem[b]).wait()
                pltpu.async_copy(rows[b], shared_agg.at[dst_v.at[j]],
                                 ssem[b], add=True)

                @pl.when(j >= 2)
                def _():
                    pltpu.make_async_copy(rows[b2],
                                          shared_agg.at[dst_v.at[0]],
                                          ssem[b2]).wait()

                @pl.when(j + 2 < C)
                def _():
                    pltpu.async_copy(x_hbm.at[src_v.at[j + 2]], rows[b2],
                                     gsem[b2])

        @pl.when((j % 2) == cid)
        def _():
            pltpu.async_copy(ones_v, shared_cnt.at[dst_v.at[j]], csem,
                             add=True)
        return carry
    lax.fori_loop(0, C, _chunk, 0)

    pltpu.make_async_copy(rows[(C - 2) % 4], shared_agg.at[dst_v.at[0]],
                          ssem[(C - 2) % 4]).wait()
    pltpu.make_async_copy(rows[(C - 1) % 4], shared_agg.at[dst_v.at[0]],
                          ssem[(C - 1) % 4]).wait()

    n_counts = jnp.where(cid == 0, (C + 1) // 2, C // 2)

    def _drain(i, carry):
        pltpu.make_async_copy(ones_v, shared_cnt.at[dst_v.at[0]],
                              csem).wait()
        return carry
    lax.fori_loop(0, n_counts, _drain, 0)
    plsc.subcore_barrier()

    pltpu.sync_copy(shared_agg.at[pl.ds(base, RPW)],
                    agg_out.at[cid, pl.ds(base, RPW)])
    pltpu.sync_copy(shared_cnt.at[pl.ds(base, RPW)],
                    cnt_out.at[cid, pl.ds(base, RPW)])


def _sc_aggregate(xs, src0, src1, dst):
    mesh = plsc.VectorSubcoreMesh(core_axis_name="c", subcore_axis_name="s")
    kern = pl.kernel(
        _sc_aggregate_body,
        out_type=[
            jax.ShapeDtypeStruct((NC, N_PAD, DH), jnp.float32),
            jax.ShapeDtypeStruct((NC, N_PAD), jnp.float32),
        ],
        mesh=mesh,
        scratch_types=[
            pltpu.VMEM((C, K), jnp.int32),
            pltpu.VMEM((C, K), jnp.int32),
            pltpu.VMEM((K, DH), jnp.float32),
            pltpu.VMEM((K, DH), jnp.float32),
            pltpu.VMEM((K, DH), jnp.float32),
            pltpu.VMEM((K, DH), jnp.float32),
            pltpu.VMEM((K,), jnp.float32),
            pltpu.VMEM_SHARED((N_PAD, DH), jnp.float32),
            pltpu.VMEM_SHARED((N_PAD,), jnp.float32),
        ] + [pltpu.SemaphoreType.DMA] * 9,
        compiler_params=pltpu.CompilerParams(use_tc_tiling_on_sc=False),
    )
    return kern(xs, src0, src1, dst)


def _tc_layer1_body(p, c, x, wl, bl, wr, o_ref):
    cnt = jnp.maximum(c[0] + c[1], 1.0)
    mean = jnp.concatenate([p[0], p[1]], axis=1) / cnt
    h = (jnp.dot(mean, wl[...], preferred_element_type=jnp.float32)
         + bl[...]
         + jnp.dot(x[...], wr[...], preferred_element_type=jnp.float32))
    o_ref[...] = jnp.maximum(h, 0.0)


def _tc_layer2_body(p, c, x, wl, bl, wr, wlin, blin, o_ref):
    cnt = jnp.maximum(c[0] + c[1], 1.0)
    mean = jnp.concatenate([p[0], p[1]], axis=1) / cnt
    h = (jnp.dot(mean, wl[...], preferred_element_type=jnp.float32)
         + bl[...]
         + jnp.dot(x[...], wr[...], preferred_element_type=jnp.float32))
    h = jnp.maximum(h, 0.0)
    o_ref[...] = (jnp.dot(h, wlin[...], preferred_element_type=jnp.float32)
                  + blin[...])


_BN = 2000


def _row_spec():
    return pl.BlockSpec((_BN, D), lambda i: (i, 0))


def _agg_spec():
    return pl.BlockSpec((NC, _BN, DH), lambda i: (0, i, 0))


def _cnt_spec():
    return pl.BlockSpec((NC, _BN, 1), lambda i: (0, i, 0))


def _w_spec():
    return pl.BlockSpec((D, D), lambda i: (0, 0))


def _b_spec():
    return pl.BlockSpec((1, D), lambda i: (0, 0))


def _tc_layer1(p, c, x, wl, bl, wr):
    return pl.pallas_call(
        _tc_layer1_body,
        grid=(N // _BN,),
        in_specs=[_agg_spec(), _cnt_spec(),
                  _row_spec(), _w_spec(), _b_spec(), _w_spec()],
        out_specs=_row_spec(),
        out_shape=jax.ShapeDtypeStruct((N, D), jnp.float32),
    )(p, c, x, wl, bl, wr)


def _tc_layer2(p, c, x, wl, bl, wr, wlin, blin):
    return pl.pallas_call(
        _tc_layer2_body,
        grid=(N // _BN,),
        in_specs=[_agg_spec(), _cnt_spec(),
                  _row_spec(), _w_spec(), _b_spec(), _w_spec(),
                  _w_spec(), _b_spec()],
        out_specs=_row_spec(),
        out_shape=jax.ShapeDtypeStruct((N, D), jnp.float32),
    )(p, c, x, wl, bl, wr, wlin, blin)


def kernel(x, edge_index, W1l, b1l, W1r, W2l, b2l, W2r, Wlin, blin):
    x = x.astype(jnp.float32)

    srcf = edge_index[0].reshape(NS, EPW)
    dstf = edge_index[1].reshape(NS, EPW)
    pad_pos = jnp.arange(NS * PADE, dtype=jnp.int32).reshape(NS, PADE)
    pad_src = (pad_pos * 613) % N
    pad_dst = N + (pad_pos % (N_PAD - N))
    srcp = jnp.concatenate([srcf, pad_src], axis=1)
    src0 = (2 * srcp).reshape(NS, C, K)
    src1 = (2 * srcp + 1).reshape(NS, C, K)
    dst = jnp.concatenate([dstf, pad_dst], axis=1).reshape(NS, C, K)

    agg1, cnt1 = _sc_aggregate(x.reshape(2 * N, DH), src0, src1, dst)
    cnt1 = cnt1[..., None]

    h = _tc_layer1(agg1, cnt1, x, W1l, b1l[None, :], W1r)

    agg2, _ = _sc_aggregate(h.reshape(2 * N, DH), src0, src1, dst)
    return _tc_layer2(agg2, cnt1, h,
                      W2l, b2l[None, :], W2r, Wlin, blin[None, :])
